# baseline (speedup 1.0000x reference)
import functools

import jax
import jax.numpy as jnp
from jax import lax
from jax.experimental import pallas as pl
from jax.experimental.pallas import tpu as pltpu
from jax.experimental.pallas import tpu_sc as plsc

N = 10000
E = 160000
D = 128
A = 4
H = 64

NC = 2
NS = 16
NW = NC * NS
CH = 128
NCHUNKP = 1280
RPT = N // NS

F32 = jnp.float32


def _dot(a, b):
    return jnp.dot(a, b, preferred_element_type=F32)


def _sig(x):
    return 1.0 / (1.0 + jnp.exp(-x))


EP = 163840
CPW = EP // NW // CH
PF = 4
EPW = EP // NW


def _gather_body(nodes_hbm, s_hbm, r_hbm, gs_hbm, gr_hbm,
                 sidx_v, ridx_v,
                 sr0, sr1, sr2, sr3, rr0, rr1, rr2, rr3,
                 sg0, sg1, sg2, sg3, sw0, sw1, sw2, sw3):
    c = lax.axis_index("c")
    s = lax.axis_index("s")
    wid = s * NC + c
    wbase = wid * EPW
    srows = [sr0, sr1, sr2, sr3]
    rrows = [rr0, rr1, rr2, rr3]
    sg = [sg0, sg1, sg2, sg3]
    sw = [sw0, sw1, sw2, sw3]

    pltpu.sync_copy(s_hbm.at[pl.ds(wid * CPW, CPW)], sidx_v)
    pltpu.sync_copy(r_hbm.at[pl.ds(wid * CPW, CPW)], ridx_v)

    def fire_g(kk, b):
        pltpu.async_copy(nodes_hbm.at[sidx_v.at[kk]], srows[b], sg[b])
        pltpu.async_copy(nodes_hbm.at[ridx_v.at[kk]], rrows[b], sg[b])

    def wait_g(b):
        pltpu.make_async_copy(nodes_hbm.at[sidx_v.at[0]], srows[b], sg[b]).wait()
        pltpu.make_async_copy(nodes_hbm.at[ridx_v.at[0]], rrows[b], sg[b]).wait()

    def fire_w(kk, b):
        off = wbase + kk * CH
        pltpu.async_copy(srows[b], gs_hbm.at[pl.ds(off, CH)], sw[b])
        pltpu.async_copy(rrows[b], gr_hbm.at[pl.ds(off, CH)], sw[b])

    def wait_w(b):
        pltpu.make_async_copy(srows[b], gs_hbm.at[pl.ds(0, CH)], sw[b]).wait()
        pltpu.make_async_copy(rrows[b], gr_hbm.at[pl.ds(0, CH)], sw[b]).wait()

    for b in range(PF):
        fire_g(b, b)

    def body(j, carry):
        for b in range(PF):
            kk = j * PF + b
            wait_g(b)
            fire_w(kk, b)
            wait_w(b)
            fire_g(kk + PF, b)
        return carry

    lax.fori_loop(0, CPW // PF - 1, body, 0)
    for b in range(PF):
        wait_g(b)
        fire_w(CPW - PF + b, b)
    for b in range(PF):
        wait_w(b)


@jax.jit
def _gather(nodes, s2, r2):
    return pl.kernel(
        _gather_body,
        mesh=plsc.VectorSubcoreMesh(core_axis_name="c", subcore_axis_name="s"),
        compiler_params=pltpu.CompilerParams(use_tc_tiling_on_sc=False),
        out_type=[
            jax.ShapeDtypeStruct((EP, H), F32),
            jax.ShapeDtypeStruct((EP, H), F32),
        ],
        scratch_types=(
            [pltpu.VMEM((CPW, CH), jnp.int32)] * 2
            + [pltpu.VMEM((CH, H), F32)] * 8
            + [pltpu.SemaphoreType.DMA] * 8
        ),
    )(nodes, s2, r2)


NPAD = N + 16


def _scatter_body(msg_hbm, r_hbm, zeros_hbm, out_hbm,
                  idx_v, r0, r1, r2, r3, agg_sh,
                  sl0, sl1, sl2, sl3, sc0, sc1, sc2, sc3):
    c = lax.axis_index("c")
    s = lax.axis_index("s")
    wid = s * NC + c
    wbase = wid * EPW
    rows = [r0, r1, r2, r3]
    sl = [sl0, sl1, sl2, sl3]
    sc = [sc0, sc1, sc2, sc3]

    pltpu.sync_copy(zeros_hbm.at[pl.ds(s * RPT, RPT)],
                    agg_sh.at[pl.ds(s * RPT, RPT)])
    pltpu.sync_copy(r_hbm.at[pl.ds(wid * CPW, CPW)], idx_v)
    plsc.subcore_barrier()

    def fire_l(kk, b):
        pltpu.async_copy(msg_hbm.at[pl.ds(wbase + kk * CH, CH)], rows[b], sl[b])

    def wait_l(b):
        pltpu.make_async_copy(msg_hbm.at[pl.ds(0, CH)], rows[b], sl[b]).wait()

    def fire_s(kk, b):
        pltpu.async_copy(rows[b], agg_sh.at[idx_v.at[kk]], sc[b], add=True)

    def wait_s(b):
        pltpu.make_async_copy(rows[b], agg_sh.at[idx_v.at[0]], sc[b]).wait()

    for b in range(PF):
        fire_l(b, b)

    def body(j, carry):
        for b in range(PF):
            kk = j * PF + b
            wait_l(b)
            fire_s(kk, b)
            wait_s(b)
            fire_l(kk + PF, b)
        return carry

    lax.fori_loop(0, CPW // PF - 1, body, 0)
    for b in range(PF):
        wait_l(b)
        fire_s(CPW - PF + b, b)
    for b in range(PF):
        wait_s(b)

    plsc.subcore_barrier()
    pltpu.sync_copy(agg_sh.at[pl.ds(s * RPT, RPT)],
                    out_hbm.at[c].at[pl.ds(s * RPT, RPT)])


@jax.jit
def _scatter(msg, r2, zeros):
    return pl.kernel(
        _scatter_body,
        mesh=plsc.VectorSubcoreMesh(core_axis_name="c", subcore_axis_name="s"),
        compiler_params=pltpu.CompilerParams(use_tc_tiling_on_sc=False),
        out_type=jax.ShapeDtypeStruct((NC, N, H), F32),
        scratch_types=(
            [pltpu.VMEM((CPW, CH), jnp.int32)]
            + [pltpu.VMEM((CH, H), F32)] * 4
            + [pltpu.VMEM_SHARED((NPAD, H), F32)]
            + [pltpu.SemaphoreType.DMA] * 8
        ),
    )(msg, r2, zeros)


def _embed_k(x_ref, na_ref, we_ref, ve_ref, out_ref):
    out_ref[...] = _dot(x_ref[...], we_ref[...]) * _dot(na_ref[...], ve_ref[...])


def _embed(x, na, We, Ve):
    return pl.pallas_call(
        _embed_k,
        out_shape=jax.ShapeDtypeStruct((N, H), F32),
    )(x, na, We, Ve)


BM = 5120


def _msg_k(gs_ref, gr_ref, ea_ref, w0a_ref, w0b_ref, v0_ref, w1_ref, v1_ref,
           out_ref):
    h = _dot(gs_ref[...], w0a_ref[...]) + _dot(gr_ref[...], w0b_ref[...])
    h = h * _dot(ea_ref[...], v0_ref[...])
    m = h[:, :H] * _sig(h[:, H:])
    h2 = _dot(m, w1_ref[...]) * _dot(ea_ref[...], v1_ref[...])
    out_ref[...] = h2[:, :H] * _sig(h2[:, H:])


def _messages(gs, gr, ea, W0a, W0b, V0, W1, V1):
    grid = EP // BM
    blk = lambda r, c: pl.BlockSpec((r, c), lambda i: (i, 0))
    wblk = lambda r, c: pl.BlockSpec((r, c), lambda i: (0, 0))
    return pl.pallas_call(
        _msg_k,
        grid=(grid,),
        in_specs=[
            blk(BM, H), blk(BM, H), blk(BM, A),
            wblk(H, 2 * H), wblk(H, 2 * H), wblk(A, 2 * H),
            wblk(H, 2 * H), wblk(A, 2 * H),
        ],
        out_specs=blk(BM, H),
        out_shape=jax.ShapeDtypeStruct((EP, H), F32),
    )(gs, gr, ea, W0a, W0b, V0, W1, V1)


def _update_k(nodes_ref, agg_ref, na_ref, wa_ref, wb_ref, v0_ref, w1_ref,
              v1_ref, out_ref):
    agg = agg_ref[0] + agg_ref[1]
    h = _dot(nodes_ref[...], wa_ref[...]) + _dot(agg, wb_ref[...])
    h = h * _dot(na_ref[...], v0_ref[...])
    u = h[:, :H] * _sig(h[:, H:])
    upd = _dot(u, w1_ref[...]) * _dot(na_ref[...], v1_ref[...])
    out_ref[...] = nodes_ref[...] + upd


def _update(nodes, agg2, na, Wa, Wb, V0, W1, V1):
    return pl.pallas_call(
        _update_k,
        out_shape=jax.ShapeDtypeStruct((N, H), F32),
    )(nodes, agg2, na, Wa, Wb, V0, W1, V1)


def _dec_k(nodes_ref, na_ref, wp_ref, vp_ref, wpp_ref, vpp_ref, wq_ref,
           wo_ref, out_ref):
    h = _dot(nodes_ref[...], wp_ref[...]) * _dot(na_ref[...], vp_ref[...])
    nd = h[:, :H] * _sig(h[:, H:])
    nd = _dot(nd, wpp_ref[...]) * _dot(na_ref[...], vpp_ref[...])
    g = jnp.sum(nd, axis=0, keepdims=True) * (1.0 / N)
    h2 = _dot(g, wq_ref[...])
    f = h2[:, :H] * _sig(h2[:, H:])
    out_ref[...] = _dot(f, wo_ref[...])


def _decode(nodes, na, Wp, Vp, Wpp, Vpp, Wq, Wo):
    return pl.pallas_call(
        _dec_k,
        out_shape=jax.ShapeDtypeStruct((1, 1), F32),
    )(nodes, na, Wp, Vp, Wpp, Vpp, Wq, Wo)


def kernel(x, node_attr, edge_attr, We, Ve, Wm0, Vm0, Wm1, Vm1, Wu0, Vu0,
           Wu1, Vu1, Wp, Vp, Wpp, Vpp, Wq, Wo, edge_index):
    npad = EP - E
    senders = jnp.concatenate(
        [edge_index[0], jnp.zeros((npad,), jnp.int32)]).reshape(NCHUNKP, CH)
    receivers_g = jnp.concatenate(
        [edge_index[1], jnp.zeros((npad,), jnp.int32)]).reshape(NCHUNKP, CH)
    receivers_s = jnp.concatenate(
        [edge_index[1], jnp.full((npad,), N, jnp.int32)]).reshape(NCHUNKP, CH)
    ea_p = jnp.concatenate([edge_attr, jnp.zeros((npad, A), F32)])
    zeros = jnp.zeros((N, H), F32)

    nodes = _embed(x, node_attr, We, Ve)
    num_layers = Wm0.shape[0]
    for l in range(num_layers):
        gs, gr = _gather(nodes, senders, receivers_g)
        msg = _messages(gs, gr, ea_p,
                        Wm0[l, :H], Wm0[l, H:], Vm0[l], Wm1[l], Vm1[l])
        agg2 = _scatter(msg, receivers_s, zeros)
        nodes = _update(nodes, agg2, node_attr,
                        Wu0[l, :H], Wu0[l, H:], Vu0[l], Wu1[l], Vu1[l])
    return _decode(nodes, node_attr, Wp, Vp, Wpp, Vpp, Wq, Wo)


# speedup vs baseline: 1.0271x; 1.0271x over previous
import functools

import jax
import jax.numpy as jnp
from jax import lax
from jax.experimental import pallas as pl
from jax.experimental.pallas import tpu as pltpu
from jax.experimental.pallas import tpu_sc as plsc

N = 10000
E = 160000
D = 128
A = 4
H = 64

NC = 2
NS = 16
NW = NC * NS
CH = 128
NCHUNKP = 1280
RPT = N // NS

F32 = jnp.float32


def _dot(a, b):
    return jnp.dot(a, b, preferred_element_type=F32)


def _sig(x):
    return 1.0 / (1.0 + jnp.exp(-x))


EP = 163840
CPW = EP // NW // CH
PF = 4
EPW = EP // NW


def _gather_body(nodes_hbm, s_hbm, r_hbm, gs_hbm, gr_hbm,
                 sidx_v, ridx_v,
                 sr0, sr1, sr2, sr3, rr0, rr1, rr2, rr3,
                 sg0, sg1, sg2, sg3, sw0, sw1, sw2, sw3):
    c = lax.axis_index("c")
    s = lax.axis_index("s")
    wid = s * NC + c
    wbase = wid * EPW
    srows = [sr0, sr1, sr2, sr3]
    rrows = [rr0, rr1, rr2, rr3]
    sg = [sg0, sg1, sg2, sg3]
    sw = [sw0, sw1, sw2, sw3]

    pltpu.sync_copy(s_hbm.at[pl.ds(wbase, EPW)], sidx_v)
    pltpu.sync_copy(r_hbm.at[pl.ds(wbase, EPW)], ridx_v)

    def fire_g(kk, b):
        pltpu.async_copy(nodes_hbm.at[sidx_v.at[pl.ds(kk * CH, CH)]],
                         srows[b], sg[b])
        pltpu.async_copy(nodes_hbm.at[ridx_v.at[pl.ds(kk * CH, CH)]],
                         rrows[b], sg[b])

    def wait_g(b):
        pltpu.make_async_copy(nodes_hbm.at[sidx_v.at[pl.ds(0, CH)]],
                              srows[b], sg[b]).wait()
        pltpu.make_async_copy(nodes_hbm.at[ridx_v.at[pl.ds(0, CH)]],
                              rrows[b], sg[b]).wait()

    def fire_w(kk, b):
        off = wbase + kk * CH
        pltpu.async_copy(srows[b], gs_hbm.at[pl.ds(off, CH)], sw[b])
        pltpu.async_copy(rrows[b], gr_hbm.at[pl.ds(off, CH)], sw[b])

    def wait_w(b):
        pltpu.make_async_copy(srows[b], gs_hbm.at[pl.ds(0, CH)], sw[b]).wait()
        pltpu.make_async_copy(rrows[b], gr_hbm.at[pl.ds(0, CH)], sw[b]).wait()

    for b in range(PF):
        fire_g(b, b)

    def body(j, carry):
        for b in range(PF):
            kk = j * PF + b
            wait_g(b)
            fire_w(kk, b)
            wait_w(b)
            fire_g(kk + PF, b)
        return carry

    lax.fori_loop(0, CPW // PF - 1, body, 0)
    for b in range(PF):
        wait_g(b)
        fire_w(CPW - PF + b, b)
    for b in range(PF):
        wait_w(b)


@jax.jit
def _gather(nodes, s2, r2):
    return pl.kernel(
        _gather_body,
        mesh=plsc.VectorSubcoreMesh(core_axis_name="c", subcore_axis_name="s"),
        compiler_params=pltpu.CompilerParams(use_tc_tiling_on_sc=False),
        out_type=[
            jax.ShapeDtypeStruct((EP, H), F32),
            jax.ShapeDtypeStruct((EP, H), F32),
        ],
        scratch_types=(
            [pltpu.VMEM((EPW,), jnp.int32)] * 2
            + [pltpu.VMEM((CH, H), F32)] * 8
            + [pltpu.SemaphoreType.DMA] * 8
        ),
    )(nodes, s2, r2)


NPAD = N + 16


def _scatter_body(msg_hbm, r_hbm, zeros_hbm, out_hbm,
                  idx_v, r0, r1, r2, r3, agg_sh,
                  sl0, sl1, sl2, sl3, sc0, sc1, sc2, sc3):
    c = lax.axis_index("c")
    s = lax.axis_index("s")
    wid = s * NC + c
    wbase = wid * EPW
    rows = [r0, r1, r2, r3]
    sl = [sl0, sl1, sl2, sl3]
    sc = [sc0, sc1, sc2, sc3]

    pltpu.sync_copy(zeros_hbm.at[pl.ds(s * RPT, RPT)],
                    agg_sh.at[pl.ds(s * RPT, RPT)])
    for k in range(CPW):
        pltpu.async_copy(r_hbm.at[pl.ds(wbase + k * CH, CH)], idx_v.at[k],
                         sl0)
    for k in range(CPW):
        pltpu.make_async_copy(r_hbm.at[pl.ds(0, CH)], idx_v.at[0], sl0).wait()
    plsc.subcore_barrier()

    def fire_l(kk, b):
        pltpu.async_copy(msg_hbm.at[pl.ds(wbase + kk * CH, CH)], rows[b], sl[b])

    def wait_l(b):
        pltpu.make_async_copy(msg_hbm.at[pl.ds(0, CH)], rows[b], sl[b]).wait()

    def fire_s(kk, b):
        pltpu.async_copy(rows[b], agg_sh.at[idx_v.at[kk]], sc[b], add=True)

    def wait_s(b):
        pltpu.make_async_copy(rows[b], agg_sh.at[idx_v.at[0]], sc[b]).wait()

    for b in range(PF):
        fire_l(b, b)

    def body(j, carry):
        for b in range(PF):
            kk = j * PF + b
            wait_l(b)
            fire_s(kk, b)
            wait_s(b)
            fire_l(kk + PF, b)
        return carry

    lax.fori_loop(0, CPW // PF - 1, body, 0)
    for b in range(PF):
        wait_l(b)
        fire_s(CPW - PF + b, b)
    for b in range(PF):
        wait_s(b)

    plsc.subcore_barrier()
    pltpu.sync_copy(agg_sh.at[pl.ds(s * RPT, RPT)],
                    out_hbm.at[c].at[pl.ds(s * RPT, RPT)])


@jax.jit
def _scatter(msg, r2, zeros):
    return pl.kernel(
        _scatter_body,
        mesh=plsc.VectorSubcoreMesh(core_axis_name="c", subcore_axis_name="s"),
        compiler_params=pltpu.CompilerParams(use_tc_tiling_on_sc=False),
        out_type=jax.ShapeDtypeStruct((NC, N, H), F32),
        scratch_types=(
            [pltpu.VMEM((CPW, CH), jnp.int32)]
            + [pltpu.VMEM((CH, H), F32)] * 4
            + [pltpu.VMEM_SHARED((NPAD, H), F32)]
            + [pltpu.SemaphoreType.DMA] * 8
        ),
    )(msg, r2, zeros)


def _embed_k(x_ref, na_ref, we_ref, ve_ref, out_ref):
    out_ref[...] = _dot(x_ref[...], we_ref[...]) * _dot(na_ref[...], ve_ref[...])


def _embed(x, na, We, Ve):
    return pl.pallas_call(
        _embed_k,
        out_shape=jax.ShapeDtypeStruct((N, H), F32),
    )(x, na, We, Ve)


BM = 5120


def _msg_k(gs_ref, gr_ref, ea_ref, w0a_ref, w0b_ref, v0_ref, w1_ref, v1_ref,
           out_ref):
    h = _dot(gs_ref[...], w0a_ref[...]) + _dot(gr_ref[...], w0b_ref[...])
    h = h * _dot(ea_ref[...], v0_ref[...])
    m = h[:, :H] * _sig(h[:, H:])
    h2 = _dot(m, w1_ref[...]) * _dot(ea_ref[...], v1_ref[...])
    out_ref[...] = h2[:, :H] * _sig(h2[:, H:])


def _messages(gs, gr, ea, W0a, W0b, V0, W1, V1):
    grid = EP // BM
    blk = lambda r, c: pl.BlockSpec((r, c), lambda i: (i, 0))
    wblk = lambda r, c: pl.BlockSpec((r, c), lambda i: (0, 0))
    return pl.pallas_call(
        _msg_k,
        grid=(grid,),
        in_specs=[
            blk(BM, H), blk(BM, H), blk(BM, A),
            wblk(H, 2 * H), wblk(H, 2 * H), wblk(A, 2 * H),
            wblk(H, 2 * H), wblk(A, 2 * H),
        ],
        out_specs=blk(BM, H),
        out_shape=jax.ShapeDtypeStruct((EP, H), F32),
    )(gs, gr, ea, W0a, W0b, V0, W1, V1)


def _update_k(nodes_ref, agg_ref, na_ref, wa_ref, wb_ref, v0_ref, w1_ref,
              v1_ref, out_ref):
    agg = agg_ref[0] + agg_ref[1]
    h = _dot(nodes_ref[...], wa_ref[...]) + _dot(agg, wb_ref[...])
    h = h * _dot(na_ref[...], v0_ref[...])
    u = h[:, :H] * _sig(h[:, H:])
    upd = _dot(u, w1_ref[...]) * _dot(na_ref[...], v1_ref[...])
    out_ref[...] = nodes_ref[...] + upd


def _update(nodes, agg2, na, Wa, Wb, V0, W1, V1):
    return pl.pallas_call(
        _update_k,
        out_shape=jax.ShapeDtypeStruct((N, H), F32),
    )(nodes, agg2, na, Wa, Wb, V0, W1, V1)


def _dec_k(nodes_ref, na_ref, wp_ref, vp_ref, wpp_ref, vpp_ref, wq_ref,
           wo_ref, out_ref):
    h = _dot(nodes_ref[...], wp_ref[...]) * _dot(na_ref[...], vp_ref[...])
    nd = h[:, :H] * _sig(h[:, H:])
    nd = _dot(nd, wpp_ref[...]) * _dot(na_ref[...], vpp_ref[...])
    g = jnp.sum(nd, axis=0, keepdims=True) * (1.0 / N)
    h2 = _dot(g, wq_ref[...])
    f = h2[:, :H] * _sig(h2[:, H:])
    out_ref[...] = _dot(f, wo_ref[...])


def _decode(nodes, na, Wp, Vp, Wpp, Vpp, Wq, Wo):
    return pl.pallas_call(
        _dec_k,
        out_shape=jax.ShapeDtypeStruct((1, 1), F32),
    )(nodes, na, Wp, Vp, Wpp, Vpp, Wq, Wo)


def kernel(x, node_attr, edge_attr, We, Ve, Wm0, Vm0, Wm1, Vm1, Wu0, Vu0,
           Wu1, Vu1, Wp, Vp, Wpp, Vpp, Wq, Wo, edge_index):
    npad = EP - E
    senders = jnp.concatenate([edge_index[0], jnp.zeros((npad,), jnp.int32)])
    receivers_g = jnp.concatenate(
        [edge_index[1], jnp.zeros((npad,), jnp.int32)])
    receivers_s = jnp.concatenate(
        [edge_index[1], jnp.full((npad,), N, jnp.int32)])
    zeros = jnp.zeros((N, H), F32)

    nodes = _embed(x, node_attr, We, Ve)
    num_layers = Wm0.shape[0]
    for l in range(num_layers):
        gs, gr = _gather(nodes, senders, receivers_g)
        msg = _messages(gs, gr, edge_attr,
                        Wm0[l, :H], Wm0[l, H:], Vm0[l], Wm1[l], Vm1[l])
        agg2 = _scatter(msg, receivers_s, zeros)
        nodes = _update(nodes, agg2, node_attr,
                        Wu0[l, :H], Wu0[l, H:], Vu0[l], Wu1[l], Vu1[l])
    return _decode(nodes, node_attr, Wp, Vp, Wpp, Vpp, Wq, Wo)
